# Initial kernel scaffold; baseline (speedup 1.0000x reference)
import jax
import jax.numpy as jnp
from jax.experimental import pallas as pl


def kernel(x, edge_index, edge_attr, W1, b1, W2, b2, W3, b3):
    raise NotImplementedError("write your pallas kernel here")


# speedup vs baseline: 11.4163x; 11.4163x over previous
import functools

import jax
import jax.numpy as jnp
from jax import lax
from jax.experimental import pallas as pl
from jax.experimental.pallas import tpu as pltpu
from jax.experimental.pallas import tpu_sc as plsc

N = 10000
E = 320000
NC = 2
NS = 16
NW = NC * NS
EPW = E // NW
K = 80
NCH = EPW // K
NP = 10240
RPT = NP // NS

_mesh = plsc.VectorSubcoreMesh(
    core_axis_name="c", subcore_axis_name="s", num_cores=NC, num_subcores=NS)
_sc_params = pltpu.CompilerParams(needs_layout_passes=False)


def _deg_body(col_h, ew_h, degp_h, col_v, ew_v, deg_v, sem):
  c = lax.axis_index("c")
  s = lax.axis_index("s")
  wid = s * NC + c
  base = wid * EPW
  cp = pltpu.async_copy(col_h.at[pl.ds(base, EPW)], col_v, sem)
  cp2 = pltpu.async_copy(ew_h.at[pl.ds(base, EPW)], ew_v, sem)

  def zero(i, _):
    deg_v[pl.ds(i * 16, 16)] = jnp.zeros((16,), jnp.float32)
    return 0
  lax.fori_loop(0, N // 16, zero, 0)
  cp.wait()
  cp2.wait()

  def acc(i, _):
    idx = col_v[pl.ds(i * 16, 16)]
    w = ew_v[pl.ds(i * 16, 16)]
    plsc.addupdate_scatter(deg_v, [idx], w)
    return 0
  lax.fori_loop(0, EPW // 16, acc, 0)
  pltpu.sync_copy(deg_v, degp_h.at[pl.ds(wid * N, N)])


_deg_call = pl.kernel(
    _deg_body,
    out_type=jax.ShapeDtypeStruct((NW * N,), jnp.float32),
    mesh=_mesh,
    scratch_types=[
        pltpu.VMEM((EPW,), jnp.int32),
        pltpu.VMEM((EPW,), jnp.float32),
        pltpu.VMEM((N,), jnp.float32),
        pltpu.SemaphoreType.DMA,
    ],
    compiler_params=_sc_params,
)


def _agg_body(d, g_h, row_h, col_h, ew_h, zero_h, out_h,
              row_v, col_v, ew_v, rows_v, acc, sem):
  c = lax.axis_index("c")
  s = lax.axis_index("s")
  wid = s * NC + c
  r0 = s * RPT
  pltpu.sync_copy(zero_h.at[pl.ds(r0, RPT)], acc.at[pl.ds(r0, RPT)])
  plsc.subcore_barrier()

  def chunk(i, _):
    base = wid * EPW + i * K
    cpr = pltpu.async_copy(row_h.at[pl.ds(base, K)], row_v, sem)
    cpc = pltpu.async_copy(col_h.at[pl.ds(base, K)], col_v, sem)
    cpw = pltpu.async_copy(ew_h.at[pl.ds(base, K)], ew_v, sem)
    cpr.wait()
    cpc.wait()
    cpw.wait()
    pltpu.async_copy(g_h.at[row_v], rows_v, sem).wait()

    def group(gi, _):
      wv = ew_v[pl.ds(gi * 16, 16)]
      for lane in range(16):
        bv = jnp.full((16,), wv[lane], jnp.float32)
        e = gi * 16 + lane
        for j in range(d // 16):
          sl = pl.ds(j * 16, 16)
          rows_v[e, sl] = rows_v[e, sl] * bv
      return 0
    lax.fori_loop(0, K // 16, group, 0)
    pltpu.sync_copy(rows_v, acc.at[col_v], add=True)
    return 0
  lax.fori_loop(0, NCH, chunk, 0)

  plsc.subcore_barrier()
  pltpu.sync_copy(acc.at[pl.ds(r0, RPT)], out_h.at[c, pl.ds(r0, RPT)])


def _make_agg(d):
  return pl.kernel(
      functools.partial(_agg_body, d),
      out_type=jax.ShapeDtypeStruct((NC, NP, d), jnp.float32),
      mesh=_mesh,
      scratch_types=[
          pltpu.VMEM((K,), jnp.int32),
          pltpu.VMEM((K,), jnp.int32),
          pltpu.VMEM((K,), jnp.float32),
          pltpu.VMEM((K, d), jnp.float32),
          pltpu.VMEM_SHARED((NP, d), jnp.float32),
          pltpu.SemaphoreType.DMA,
      ],
      compiler_params=_sc_params,
  )


_agg128 = _make_agg(128)


_BM = 1000


def _dinv_body(degp_ref, out_ref):
  s = jnp.sum(degp_ref[...], axis=0, keepdims=True)
  out_ref[...] = lax.rsqrt(1.0 + s)


def _dinv_call(degp):
  return pl.pallas_call(
      _dinv_body,
      out_shape=jax.ShapeDtypeStruct((1, N), jnp.float32),
  )(degp)


def _pre_body(x_ref, w_ref, dinv_ref, o_ref):
  h = jnp.dot(x_ref[...], w_ref[...], preferred_element_type=jnp.float32)
  o_ref[...] = dinv_ref[...] * h


def _pre_call(x, w, dinv_col):
  dout = w.shape[1]
  return pl.pallas_call(
      _pre_body,
      grid=(N // _BM,),
      in_specs=[
          pl.BlockSpec((_BM, 128), lambda i: (i, 0)),
          pl.BlockSpec((128, dout), lambda i: (0, 0)),
          pl.BlockSpec((_BM, 1), lambda i: (i, 0)),
      ],
      out_specs=pl.BlockSpec((_BM, dout), lambda i: (i, 0)),
      out_shape=jax.ShapeDtypeStruct((N, dout), jnp.float32),
  )(x, w, dinv_col)


def _mid_body(p_ref, g_ref, dinv_ref, b_ref, w_ref, o_ref):
  t = p_ref[0] + p_ref[1] + g_ref[...]
  xn = jnp.maximum(dinv_ref[...] * t + b_ref[...], 0.0)
  h = jnp.dot(xn, w_ref[...], preferred_element_type=jnp.float32)
  o_ref[...] = dinv_ref[...] * h


def _mid_call(p, g, dinv_col, b_row, w):
  din = g.shape[1]
  dout = w.shape[1]
  return pl.pallas_call(
      _mid_body,
      grid=(N // _BM,),
      in_specs=[
          pl.BlockSpec((NC, _BM, din), lambda i: (0, i, 0)),
          pl.BlockSpec((_BM, din), lambda i: (i, 0)),
          pl.BlockSpec((_BM, 1), lambda i: (i, 0)),
          pl.BlockSpec((1, din), lambda i: (0, 0)),
          pl.BlockSpec((din, dout), lambda i: (0, 0)),
      ],
      out_specs=pl.BlockSpec((_BM, dout), lambda i: (i, 0)),
      out_shape=jax.ShapeDtypeStruct((N, dout), jnp.float32),
  )(p, g, dinv_col, b_row, w)


def _post_body(p_ref, g_ref, dinv_ref, b_ref, o_ref):
  t = p_ref[0] + p_ref[1] + g_ref[...]
  o_ref[...] = dinv_ref[...] * t + b_ref[...]


def _post_call(p, g, dinv_col, b_row):
  d = g.shape[1]
  return pl.pallas_call(
      _post_body,
      grid=(N // _BM,),
      in_specs=[
          pl.BlockSpec((NC, _BM, d), lambda i: (0, i, 0)),
          pl.BlockSpec((_BM, d), lambda i: (i, 0)),
          pl.BlockSpec((_BM, 1), lambda i: (i, 0)),
          pl.BlockSpec((1, d), lambda i: (0, 0)),
      ],
      out_specs=pl.BlockSpec((_BM, d), lambda i: (i, 0)),
      out_shape=jax.ShapeDtypeStruct((N, d), jnp.float32),
  )(p, g, dinv_col, b_row)


def kernel(x, edge_index, edge_attr, W1, b1, W2, b2, W3, b3):
  row = edge_index[0].astype(jnp.int32)
  col = edge_index[1].astype(jnp.int32)
  ew = edge_attr.astype(jnp.float32)

  degp = _deg_call(col, ew).reshape(NW, N)
  dinv_col = _dinv_call(degp).reshape(N, 1)

  z128 = jnp.zeros((NP, 128), jnp.float32)
  W3p = jnp.pad(W3, ((0, 0), (0, 126)))
  b3p = jnp.pad(b3, (0, 126))

  g1 = _pre_call(x, W1, dinv_col)
  p1 = _agg128(g1, row, col, ew, z128)[:, :N]
  g2 = _mid_call(p1, g1, dinv_col, b1.reshape(1, 128), W2)
  p2 = _agg128(g2, row, col, ew, z128)[:, :N]
  g3 = _mid_call(p2, g2, dinv_col, b2.reshape(1, 128), W3p)
  p3 = _agg128(g3, row, col, ew, z128)[:, :N]
  out = _post_call(p3, g3, dinv_col, b3p.reshape(1, 128))
  return out[:, :2]


# speedup vs baseline: 11.9542x; 1.0471x over previous
import functools

import jax
import jax.numpy as jnp
from jax import lax
from jax.experimental import pallas as pl
from jax.experimental.pallas import tpu as pltpu
from jax.experimental.pallas import tpu_sc as plsc

N = 10000
E = 320000
NC = 2
NS = 16
NW = NC * NS
EPW = E // NW
K = 80
NCH = EPW // K
NP = 10240
RPT = NP // NS

_mesh = plsc.VectorSubcoreMesh(
    core_axis_name="c", subcore_axis_name="s", num_cores=NC, num_subcores=NS)
_sc_params = pltpu.CompilerParams(needs_layout_passes=False)


def _deg_body(col_h, ew_h, degp_h, col_v, ew_v, deg_v, sem):
  c = lax.axis_index("c")
  s = lax.axis_index("s")
  wid = s * NC + c
  base = wid * EPW
  cp = pltpu.async_copy(col_h.at[pl.ds(base, EPW)], col_v, sem)
  cp2 = pltpu.async_copy(ew_h.at[pl.ds(base, EPW)], ew_v, sem)

  def zero(i, _):
    deg_v[pl.ds(i * 16, 16)] = jnp.zeros((16,), jnp.float32)
    return 0
  lax.fori_loop(0, N // 16, zero, 0)
  cp.wait()
  cp2.wait()

  def acc(i, _):
    idx = col_v[pl.ds(i * 16, 16)]
    w = ew_v[pl.ds(i * 16, 16)]
    plsc.addupdate_scatter(deg_v, [idx], w)
    return 0
  lax.fori_loop(0, EPW // 16, acc, 0)
  pltpu.sync_copy(deg_v, degp_h.at[pl.ds(wid * N, N)])


_deg_call = pl.kernel(
    _deg_body,
    out_type=jax.ShapeDtypeStruct((NW * N,), jnp.float32),
    mesh=_mesh,
    scratch_types=[
        pltpu.VMEM((EPW,), jnp.int32),
        pltpu.VMEM((EPW,), jnp.float32),
        pltpu.VMEM((N,), jnp.float32),
        pltpu.SemaphoreType.DMA,
    ],
    compiler_params=_sc_params,
)


def _agg_body(d, g_h, row_h, col_h, ew_h, out_h,
              r0i, r1i, c0i, c1i, w0i, w1i, rows0, rows1, acc,
              g0s, g1s, s0s, s1s):
  c = lax.axis_index("c")
  s = lax.axis_index("s")
  wid = s * NC + c
  r0 = s * RPT
  nj = d // 16

  def zrow(r, _):
    for j in range(nj):
      rows0[r, pl.ds(j * 16, 16)] = jnp.zeros((16,), jnp.float32)
    return 0
  lax.fori_loop(0, K, zrow, 0)

  def zcp(k, _):
    pltpu.sync_copy(rows0, acc.at[pl.ds(r0 + k * K, K)])
    return 0
  lax.fori_loop(0, RPT // K, zcp, 0)
  plsc.subcore_barrier()

  def scale(wv_ref, rows_v):
    def group(gi, _):
      wv = wv_ref[pl.ds(gi * 16, 16)]
      for lane in range(16):
        bv = jnp.full((16,), wv[lane], jnp.float32)
        e = gi * 16 + lane
        for j in range(nj):
          sl = pl.ds(j * 16, 16)
          rows_v[e, sl] = rows_v[e, sl] * bv
      return 0
    lax.fori_loop(0, K // 16, group, 0)

  def fetch(i, riv, civ, wiv, rows_v, gsem):
    base = wid * EPW + i * K
    pltpu.sync_copy(row_h.at[pl.ds(base, K)], riv)
    pltpu.sync_copy(col_h.at[pl.ds(base, K)], civ)
    pltpu.sync_copy(ew_h.at[pl.ds(base, K)], wiv)
    return pltpu.async_copy(g_h.at[riv], rows_v, gsem)

  def pair(m, _):
    i0 = 2 * m
    i1 = i0 + 1
    cp0 = fetch(i0, r0i, c0i, w0i, rows0, g0s)
    cp1 = fetch(i1, r1i, c1i, w1i, rows1, g1s)
    cp0.wait()
    scale(w0i, rows0)
    sc0 = pltpu.async_copy(rows0, acc.at[c0i], s0s, add=True)
    cp1.wait()
    scale(w1i, rows1)
    sc1 = pltpu.async_copy(rows1, acc.at[c1i], s1s, add=True)
    sc0.wait()
    sc1.wait()
    return 0
  lax.fori_loop(0, NCH // 2, pair, 0)

  fetch(NCH - 1, r0i, c0i, w0i, rows0, g0s).wait()
  scale(w0i, rows0)
  pltpu.sync_copy(rows0, acc.at[c0i], add=True)

  plsc.subcore_barrier()
  pltpu.sync_copy(acc.at[pl.ds(r0, RPT)], out_h.at[c, pl.ds(r0, RPT)])


def _make_agg(d):
  return pl.kernel(
      functools.partial(_agg_body, d),
      out_type=jax.ShapeDtypeStruct((NC, NP, d), jnp.float32),
      mesh=_mesh,
      scratch_types=[
          pltpu.VMEM((K,), jnp.int32),
          pltpu.VMEM((K,), jnp.int32),
          pltpu.VMEM((K,), jnp.int32),
          pltpu.VMEM((K,), jnp.int32),
          pltpu.VMEM((K,), jnp.float32),
          pltpu.VMEM((K,), jnp.float32),
          pltpu.VMEM((K, d), jnp.float32),
          pltpu.VMEM((K, d), jnp.float32),
          pltpu.VMEM_SHARED((NP, d), jnp.float32),
          pltpu.SemaphoreType.DMA,
          pltpu.SemaphoreType.DMA,
          pltpu.SemaphoreType.DMA,
          pltpu.SemaphoreType.DMA,
      ],
      compiler_params=_sc_params,
  )


_agg128 = _make_agg(128)


_BM = 1000


def _dinv_body(degp_ref, out_ref):
  s = jnp.sum(degp_ref[...], axis=0, keepdims=True)
  out_ref[...] = lax.rsqrt(1.0 + s)


def _dinv_call(degp):
  return pl.pallas_call(
      _dinv_body,
      out_shape=jax.ShapeDtypeStruct((1, N), jnp.float32),
  )(degp)


def _pre_body(x_ref, w_ref, dinv_ref, o_ref):
  h = jnp.dot(x_ref[...], w_ref[...], preferred_element_type=jnp.float32)
  o_ref[...] = dinv_ref[...] * h


def _pre_call(x, w, dinv_col):
  dout = w.shape[1]
  return pl.pallas_call(
      _pre_body,
      grid=(N // _BM,),
      in_specs=[
          pl.BlockSpec((_BM, 128), lambda i: (i, 0)),
          pl.BlockSpec((128, dout), lambda i: (0, 0)),
          pl.BlockSpec((_BM, 1), lambda i: (i, 0)),
      ],
      out_specs=pl.BlockSpec((_BM, dout), lambda i: (i, 0)),
      out_shape=jax.ShapeDtypeStruct((N, dout), jnp.float32),
  )(x, w, dinv_col)


def _mid_body(p_ref, g_ref, dinv_ref, b_ref, w_ref, o_ref):
  t = p_ref[0] + p_ref[1] + g_ref[...]
  xn = jnp.maximum(dinv_ref[...] * t + b_ref[...], 0.0)
  h = jnp.dot(xn, w_ref[...], preferred_element_type=jnp.float32)
  o_ref[...] = dinv_ref[...] * h


def _mid_call(p, g, dinv_col, b_row, w):
  din = g.shape[1]
  dout = w.shape[1]
  return pl.pallas_call(
      _mid_body,
      grid=(N // _BM,),
      in_specs=[
          pl.BlockSpec((NC, _BM, din), lambda i: (0, i, 0)),
          pl.BlockSpec((_BM, din), lambda i: (i, 0)),
          pl.BlockSpec((_BM, 1), lambda i: (i, 0)),
          pl.BlockSpec((1, din), lambda i: (0, 0)),
          pl.BlockSpec((din, dout), lambda i: (0, 0)),
      ],
      out_specs=pl.BlockSpec((_BM, dout), lambda i: (i, 0)),
      out_shape=jax.ShapeDtypeStruct((N, dout), jnp.float32),
  )(p, g, dinv_col, b_row, w)


def _post_body(p_ref, g_ref, dinv_ref, b_ref, o_ref):
  t = p_ref[0] + p_ref[1] + g_ref[...]
  o_ref[...] = dinv_ref[...] * t + b_ref[...]


def _post_call(p, g, dinv_col, b_row):
  d = g.shape[1]
  return pl.pallas_call(
      _post_body,
      grid=(N // _BM,),
      in_specs=[
          pl.BlockSpec((NC, _BM, d), lambda i: (0, i, 0)),
          pl.BlockSpec((_BM, d), lambda i: (i, 0)),
          pl.BlockSpec((_BM, 1), lambda i: (i, 0)),
          pl.BlockSpec((1, d), lambda i: (0, 0)),
      ],
      out_specs=pl.BlockSpec((_BM, d), lambda i: (i, 0)),
      out_shape=jax.ShapeDtypeStruct((N, d), jnp.float32),
  )(p, g, dinv_col, b_row)


def kernel(x, edge_index, edge_attr, W1, b1, W2, b2, W3, b3):
  row = edge_index[0].astype(jnp.int32)
  col = edge_index[1].astype(jnp.int32)
  ew = edge_attr.astype(jnp.float32)

  degp = _deg_call(col, ew).reshape(NW, N)
  dinv_col = _dinv_call(degp).reshape(N, 1)

  W3p = jnp.pad(W3, ((0, 0), (0, 126)))
  b3p = jnp.pad(b3, (0, 126))

  g1 = _pre_call(x, W1, dinv_col)
  p1 = _agg128(g1, row, col, ew)[:, :N]
  g2 = _mid_call(p1, g1, dinv_col, b1.reshape(1, 128), W2)
  p2 = _agg128(g2, row, col, ew)[:, :N]
  g3 = _mid_call(p2, g2, dinv_col, b2.reshape(1, 128), W3p)
  p3 = _agg128(g3, row, col, ew)[:, :N]
  out = _post_call(p3, g3, dinv_col, b3p.reshape(1, 128))
  return out[:, :2]


# speedup vs baseline: 17.0461x; 1.4259x over previous
import functools

import jax
import jax.numpy as jnp
from jax import lax
from jax.experimental import pallas as pl
from jax.experimental.pallas import tpu as pltpu
from jax.experimental.pallas import tpu_sc as plsc

N = 10000
E = 320000
NC = 2
NS = 16
NW = NC * NS
EPW = E // NW
K = 80
NCH = EPW // K
TAIL = 16
NP = 10240
RPT = NP // NS

_mesh = plsc.VectorSubcoreMesh(
    core_axis_name="c", subcore_axis_name="s", num_cores=NC, num_subcores=NS)
_sc_params = pltpu.CompilerParams(needs_layout_passes=False)


def _deg_body(col_h, ew_h, degp_h, col_v, ew_v, deg_v, sem):
  c = lax.axis_index("c")
  s = lax.axis_index("s")
  wid = s * NC + c
  base = wid * EPW
  cp = pltpu.async_copy(col_h.at[pl.ds(base, EPW)], col_v, sem)
  cp2 = pltpu.async_copy(ew_h.at[pl.ds(base, EPW)], ew_v, sem)

  def zero(i, _):
    deg_v[pl.ds(i * 16, 16)] = jnp.zeros((16,), jnp.float32)
    return 0
  lax.fori_loop(0, N // 16, zero, 0)
  cp.wait()
  cp2.wait()

  def acc(i, _):
    idx = col_v[pl.ds(i * 16, 16)]
    w = ew_v[pl.ds(i * 16, 16)]
    plsc.addupdate_scatter(deg_v, [idx], w)
    return 0
  lax.fori_loop(0, EPW // 16, acc, 0)
  pltpu.sync_copy(deg_v, degp_h.at[pl.ds(wid * N, N)])


_deg_call = pl.kernel(
    _deg_body,
    out_type=jax.ShapeDtypeStruct((NW * N,), jnp.float32),
    mesh=_mesh,
    scratch_types=[
        pltpu.VMEM((EPW,), jnp.int32),
        pltpu.VMEM((EPW,), jnp.float32),
        pltpu.VMEM((N,), jnp.float32),
        pltpu.SemaphoreType.DMA,
    ],
    compiler_params=_sc_params,
)


def _agg_body(d, g_h, row_h, col_h, ew_h, out_h,
              col_a, ew_a, r0i, r1i, c0i, c1i,
              rows0, rows1, acc, g0s, g1s, s0s, s1s):
  c = lax.axis_index("c")
  s = lax.axis_index("s")
  wid = s * NC + c
  r0 = s * RPT
  nj = d // 16
  base = wid * EPW

  cpb = pltpu.async_copy(col_h.at[pl.ds(base, EPW)], col_a, g1s)
  cpc = pltpu.async_copy(ew_h.at[pl.ds(base, EPW)], ew_a, s0s)

  def zrow(r, _):
    for j in range(nj):
      rows0[r, pl.ds(j * 16, 16)] = jnp.zeros((16,), jnp.float32)
    return 0
  lax.fori_loop(0, K, zrow, 0)

  def zcp(k, _):
    pltpu.sync_copy(rows0, acc.at[pl.ds(r0 + k * K, K)])
    return 0
  lax.fori_loop(0, RPT // K, zcp, 0)
  cpb.wait()
  cpc.wait()
  plsc.subcore_barrier()

  def scale(i, n, civ, rows_v):
    def group(gi, _):
      e0 = i * K + gi * 16
      civ[pl.ds(gi * 16, 16)] = col_a[pl.ds(e0, 16)]
      wv = ew_a[pl.ds(e0, 16)]
      for lane in range(16):
        bv = jnp.full((16,), wv[lane], jnp.float32)
        e = gi * 16 + lane
        for j in range(nj):
          sl = pl.ds(j * 16, 16)
          rows_v[e, sl] = rows_v[e, sl] * bv
      return 0
    lax.fori_loop(0, n // 16, group, 0)

  def gather(i, n, riv, rows_v, gsem):
    pltpu.sync_copy(row_h.at[pl.ds(base + i * K, n)], riv)
    return pltpu.async_copy(g_h.at[riv], rows_v, gsem)

  def pair(m, _):
    i0 = 2 * m
    i1 = i0 + 1
    cp0 = gather(i0, K, r0i, rows0, g0s)
    cp1 = gather(i1, K, r1i, rows1, g1s)
    cp0.wait()
    scale(i0, K, c0i, rows0)
    sc0 = pltpu.async_copy(rows0, acc.at[c0i], s0s, add=True)
    cp1.wait()
    scale(i1, K, c1i, rows1)
    sc1 = pltpu.async_copy(rows1, acc.at[c1i], s1s, add=True)
    sc0.wait()
    sc1.wait()
    return 0
  lax.fori_loop(0, NCH // 2, pair, 0)

  gather(NCH - 1, K, r0i, rows0, g0s).wait()
  scale(NCH - 1, K, c0i, rows0)
  pltpu.sync_copy(rows0, acc.at[c0i], add=True)

  plsc.subcore_barrier()
  pltpu.sync_copy(acc.at[pl.ds(r0, RPT)], out_h.at[c, pl.ds(r0, RPT)])


def _make_agg(d):
  return pl.kernel(
      functools.partial(_agg_body, d),
      out_type=jax.ShapeDtypeStruct((NC, NP, d), jnp.float32),
      mesh=_mesh,
      scratch_types=[
          pltpu.VMEM((EPW,), jnp.int32),
          pltpu.VMEM((EPW,), jnp.float32),
          pltpu.VMEM((K,), jnp.int32),
          pltpu.VMEM((K,), jnp.int32),
          pltpu.VMEM((K,), jnp.int32),
          pltpu.VMEM((K,), jnp.int32),
          pltpu.VMEM((K, d), jnp.float32),
          pltpu.VMEM((K, d), jnp.float32),
          pltpu.VMEM_SHARED((NP, d), jnp.float32),
          pltpu.SemaphoreType.DMA,
          pltpu.SemaphoreType.DMA,
          pltpu.SemaphoreType.DMA,
          pltpu.SemaphoreType.DMA,
      ],
      compiler_params=_sc_params,
  )


_agg128 = _make_agg(128)


_BM = 1000


def _dinv_body(degp_ref, out_ref):
  s = jnp.sum(degp_ref[...], axis=0, keepdims=True)
  out_ref[...] = lax.rsqrt(1.0 + s)


def _dinv_call(degp):
  return pl.pallas_call(
      _dinv_body,
      out_shape=jax.ShapeDtypeStruct((1, N), jnp.float32),
  )(degp)


def _pre_body(x_ref, w_ref, dinv_ref, o_ref):
  h = jnp.dot(x_ref[...], w_ref[...], preferred_element_type=jnp.float32)
  o_ref[...] = dinv_ref[...] * h


def _pre_call(x, w, dinv_col):
  dout = w.shape[1]
  return pl.pallas_call(
      _pre_body,
      grid=(N // _BM,),
      in_specs=[
          pl.BlockSpec((_BM, 128), lambda i: (i, 0)),
          pl.BlockSpec((128, dout), lambda i: (0, 0)),
          pl.BlockSpec((_BM, 1), lambda i: (i, 0)),
      ],
      out_specs=pl.BlockSpec((_BM, dout), lambda i: (i, 0)),
      out_shape=jax.ShapeDtypeStruct((N, dout), jnp.float32),
  )(x, w, dinv_col)


def _mid_body(p_ref, g_ref, dinv_ref, b_ref, w_ref, o_ref):
  t = p_ref[0] + p_ref[1] + g_ref[...]
  xn = jnp.maximum(dinv_ref[...] * t + b_ref[...], 0.0)
  h = jnp.dot(xn, w_ref[...], preferred_element_type=jnp.float32)
  o_ref[...] = dinv_ref[...] * h


def _mid_call(p, g, dinv_col, b_row, w):
  din = g.shape[1]
  dout = w.shape[1]
  return pl.pallas_call(
      _mid_body,
      grid=(N // _BM,),
      in_specs=[
          pl.BlockSpec((NC, _BM, din), lambda i: (0, i, 0)),
          pl.BlockSpec((_BM, din), lambda i: (i, 0)),
          pl.BlockSpec((_BM, 1), lambda i: (i, 0)),
          pl.BlockSpec((1, din), lambda i: (0, 0)),
          pl.BlockSpec((din, dout), lambda i: (0, 0)),
      ],
      out_specs=pl.BlockSpec((_BM, dout), lambda i: (i, 0)),
      out_shape=jax.ShapeDtypeStruct((N, dout), jnp.float32),
  )(p, g, dinv_col, b_row, w)


def _post_body(p_ref, g_ref, dinv_ref, b_ref, o_ref):
  t = p_ref[0] + p_ref[1] + g_ref[...]
  o_ref[...] = dinv_ref[...] * t + b_ref[...]


def _post_call(p, g, dinv_col, b_row):
  d = g.shape[1]
  return pl.pallas_call(
      _post_body,
      grid=(N // _BM,),
      in_specs=[
          pl.BlockSpec((NC, _BM, d), lambda i: (0, i, 0)),
          pl.BlockSpec((_BM, d), lambda i: (i, 0)),
          pl.BlockSpec((_BM, 1), lambda i: (i, 0)),
          pl.BlockSpec((1, d), lambda i: (0, 0)),
      ],
      out_specs=pl.BlockSpec((_BM, d), lambda i: (i, 0)),
      out_shape=jax.ShapeDtypeStruct((N, d), jnp.float32),
  )(p, g, dinv_col, b_row)


def kernel(x, edge_index, edge_attr, W1, b1, W2, b2, W3, b3):
  row = edge_index[0].astype(jnp.int32)
  col = edge_index[1].astype(jnp.int32)
  ew = edge_attr.astype(jnp.float32)

  degp = _deg_call(col, ew).reshape(NW, N)
  dinv_col = _dinv_call(degp).reshape(N, 1)

  W3p = jnp.pad(W3, ((0, 0), (0, 126)))
  b3p = jnp.pad(b3, (0, 126))

  g1 = _pre_call(x, W1, dinv_col)
  p1 = _agg128(g1, row, col, ew)[:, :N]
  g2 = _mid_call(p1, g1, dinv_col, b1.reshape(1, 128), W2)
  p2 = _agg128(g2, row, col, ew)[:, :N]
  g3 = _mid_call(p2, g2, dinv_col, b2.reshape(1, 128), W3p)
  p3 = _agg128(g3, row, col, ew)[:, :N]
  out = _post_call(p3, g3, dinv_col, b3p.reshape(1, 128))
  return out[:, :2]


# speedup vs baseline: 17.3184x; 1.0160x over previous
import functools

import jax
import jax.numpy as jnp
from jax import lax
from jax.experimental import pallas as pl
from jax.experimental.pallas import tpu as pltpu
from jax.experimental.pallas import tpu_sc as plsc

N = 10000
E = 320000
NC = 2
NS = 16
NW = NC * NS
EPW = E // NW
K = 80
NCH = EPW // K
TAIL = 16
NP = 10240
RPT = NP // NS

_mesh = plsc.VectorSubcoreMesh(
    core_axis_name="c", subcore_axis_name="s", num_cores=NC, num_subcores=NS)
_sc_params = pltpu.CompilerParams(needs_layout_passes=False)


def _deg_body(col_h, ew_h, degp_h, col_v, ew_v, deg_v, sem):
  c = lax.axis_index("c")
  s = lax.axis_index("s")
  wid = s * NC + c
  base = wid * EPW
  cp = pltpu.async_copy(col_h.at[pl.ds(base, EPW)], col_v, sem)
  cp2 = pltpu.async_copy(ew_h.at[pl.ds(base, EPW)], ew_v, sem)

  def zero(i, _):
    deg_v[pl.ds(i * 16, 16)] = jnp.zeros((16,), jnp.float32)
    return 0
  lax.fori_loop(0, N // 16, zero, 0)
  cp.wait()
  cp2.wait()

  def acc(i, _):
    idx = col_v[pl.ds(i * 16, 16)]
    w = ew_v[pl.ds(i * 16, 16)]
    plsc.addupdate_scatter(deg_v, [idx], w)
    return 0
  lax.fori_loop(0, EPW // 16, acc, 0)
  pltpu.sync_copy(deg_v, degp_h.at[pl.ds(wid * N, N)])


_deg_call = pl.kernel(
    _deg_body,
    out_type=jax.ShapeDtypeStruct((NW * N,), jnp.float32),
    mesh=_mesh,
    scratch_types=[
        pltpu.VMEM((EPW,), jnp.int32),
        pltpu.VMEM((EPW,), jnp.float32),
        pltpu.VMEM((N,), jnp.float32),
        pltpu.SemaphoreType.DMA,
    ],
    compiler_params=_sc_params,
)


def _agg_body(d, g_h, row_h, col_h, ew_h, out_h,
              col_a, ew_a, rbig, c0i, c1i,
              rows0, rows1, acc, g0s, g1s, s0s, s1s):
  c = lax.axis_index("c")
  s = lax.axis_index("s")
  wid = s * NC + c
  r0 = s * RPT
  nj = d // 16
  base = wid * EPW

  cpb = pltpu.async_copy(col_h.at[pl.ds(base, EPW)], col_a, g1s)
  cpc = pltpu.async_copy(ew_h.at[pl.ds(base, EPW)], ew_a, s0s)

  def zrow(r, _):
    for j in range(nj):
      rows0[r, pl.ds(j * 16, 16)] = jnp.zeros((16,), jnp.float32)
    return 0
  lax.fori_loop(0, K, zrow, 0)

  def zcp(k, _):
    pltpu.sync_copy(rows0, acc.at[pl.ds(r0 + k * K, K)])
    return 0
  lax.fori_loop(0, RPT // K, zcp, 0)
  cpb.wait()
  cpc.wait()
  plsc.subcore_barrier()

  def scale(i, n, civ, rows_v):
    def group(gi, _):
      e0 = i * K + gi * 16
      civ[pl.ds(gi * 16, 16)] = col_a[pl.ds(e0, 16)]
      wv = ew_a[pl.ds(e0, 16)]
      for lane in range(16):
        bv = jnp.full((16,), wv[lane], jnp.float32)
        e = gi * 16 + lane
        for j in range(nj):
          sl = pl.ds(j * 16, 16)
          rows_v[e, sl] = rows_v[e, sl] * bv
      return 0
    lax.fori_loop(0, n // 16, group, 0)

  def pair(i0, roff):
    i1 = i0 + 1
    cp0 = pltpu.async_copy(g_h.at[rbig.at[pl.ds(roff, K)]], rows0, g0s)
    cp1 = pltpu.async_copy(g_h.at[rbig.at[pl.ds(roff + K, K)]], rows1, g1s)
    cp0.wait()
    scale(i0, K, c0i, rows0)
    sc0 = pltpu.async_copy(rows0, acc.at[c0i], s0s, add=True)
    cp1.wait()
    scale(i1, K, c1i, rows1)
    sc1 = pltpu.async_copy(rows1, acc.at[c1i], s1s, add=True)
    sc0.wait()
    sc1.wait()

  def group10(g, _):
    pltpu.sync_copy(row_h.at[pl.ds(base + g * 10 * K, 10 * K)], rbig)
    def p5(p, _):
      pair(g * 10 + 2 * p, 2 * p * K)
      return 0
    lax.fori_loop(0, 5, p5, 0)
    return 0
  lax.fori_loop(0, NCH // 10, group10, 0)

  tbase = (NCH // 10) * 10
  pltpu.sync_copy(row_h.at[pl.ds(base + tbase * K, 5 * K)],
                  rbig.at[pl.ds(0, 5 * K)])
  pair(tbase, 0)
  pair(tbase + 2, 2 * K)
  pltpu.async_copy(g_h.at[rbig.at[pl.ds(4 * K, K)]], rows0, g0s).wait()
  scale(tbase + 4, K, c0i, rows0)
  pltpu.sync_copy(rows0, acc.at[c0i], add=True)

  plsc.subcore_barrier()
  pltpu.sync_copy(acc.at[pl.ds(r0, RPT)], out_h.at[c, pl.ds(r0, RPT)])


def _make_agg(d):
  return pl.kernel(
      functools.partial(_agg_body, d),
      out_type=jax.ShapeDtypeStruct((NC, NP, d), jnp.float32),
      mesh=_mesh,
      scratch_types=[
          pltpu.VMEM((EPW,), jnp.int32),
          pltpu.VMEM((EPW,), jnp.float32),
          pltpu.VMEM((10 * K,), jnp.int32),
          pltpu.VMEM((K,), jnp.int32),
          pltpu.VMEM((K,), jnp.int32),
          pltpu.VMEM((K, d), jnp.float32),
          pltpu.VMEM((K, d), jnp.float32),
          pltpu.VMEM_SHARED((NP, d), jnp.float32),
          pltpu.SemaphoreType.DMA,
          pltpu.SemaphoreType.DMA,
          pltpu.SemaphoreType.DMA,
          pltpu.SemaphoreType.DMA,
      ],
      compiler_params=_sc_params,
  )


_agg128 = _make_agg(128)


_BM = 1000


def _dinv_body(degp_ref, out_ref):
  s = jnp.sum(degp_ref[...], axis=0, keepdims=True)
  out_ref[...] = lax.rsqrt(1.0 + s)


def _dinv_call(degp):
  return pl.pallas_call(
      _dinv_body,
      out_shape=jax.ShapeDtypeStruct((1, N), jnp.float32),
  )(degp)


def _pre_body(x_ref, w_ref, dinv_ref, o_ref):
  h = jnp.dot(x_ref[...], w_ref[...], preferred_element_type=jnp.float32)
  o_ref[...] = dinv_ref[...] * h


def _pre_call(x, w, dinv_col):
  dout = w.shape[1]
  return pl.pallas_call(
      _pre_body,
      grid=(N // _BM,),
      in_specs=[
          pl.BlockSpec((_BM, 128), lambda i: (i, 0)),
          pl.BlockSpec((128, dout), lambda i: (0, 0)),
          pl.BlockSpec((_BM, 1), lambda i: (i, 0)),
      ],
      out_specs=pl.BlockSpec((_BM, dout), lambda i: (i, 0)),
      out_shape=jax.ShapeDtypeStruct((N, dout), jnp.float32),
  )(x, w, dinv_col)


def _mid_body(p_ref, g_ref, dinv_ref, b_ref, w_ref, o_ref):
  t = p_ref[0] + p_ref[1] + g_ref[...]
  xn = jnp.maximum(dinv_ref[...] * t + b_ref[...], 0.0)
  h = jnp.dot(xn, w_ref[...], preferred_element_type=jnp.float32)
  o_ref[...] = dinv_ref[...] * h


def _mid_call(p, g, dinv_col, b_row, w):
  din = g.shape[1]
  dout = w.shape[1]
  return pl.pallas_call(
      _mid_body,
      grid=(N // _BM,),
      in_specs=[
          pl.BlockSpec((NC, _BM, din), lambda i: (0, i, 0)),
          pl.BlockSpec((_BM, din), lambda i: (i, 0)),
          pl.BlockSpec((_BM, 1), lambda i: (i, 0)),
          pl.BlockSpec((1, din), lambda i: (0, 0)),
          pl.BlockSpec((din, dout), lambda i: (0, 0)),
      ],
      out_specs=pl.BlockSpec((_BM, dout), lambda i: (i, 0)),
      out_shape=jax.ShapeDtypeStruct((N, dout), jnp.float32),
  )(p, g, dinv_col, b_row, w)


def _post_body(p_ref, g_ref, dinv_ref, b_ref, o_ref):
  t = p_ref[0] + p_ref[1] + g_ref[...]
  o_ref[...] = dinv_ref[...] * t + b_ref[...]


def _post_call(p, g, dinv_col, b_row):
  d = g.shape[1]
  return pl.pallas_call(
      _post_body,
      grid=(N // _BM,),
      in_specs=[
          pl.BlockSpec((NC, _BM, d), lambda i: (0, i, 0)),
          pl.BlockSpec((_BM, d), lambda i: (i, 0)),
          pl.BlockSpec((_BM, 1), lambda i: (i, 0)),
          pl.BlockSpec((1, d), lambda i: (0, 0)),
      ],
      out_specs=pl.BlockSpec((_BM, d), lambda i: (i, 0)),
      out_shape=jax.ShapeDtypeStruct((N, d), jnp.float32),
  )(p, g, dinv_col, b_row)


def kernel(x, edge_index, edge_attr, W1, b1, W2, b2, W3, b3):
  row = edge_index[0].astype(jnp.int32)
  col = edge_index[1].astype(jnp.int32)
  ew = edge_attr.astype(jnp.float32)

  degp = _deg_call(col, ew).reshape(NW, N)
  dinv_col = _dinv_call(degp).reshape(N, 1)

  W3p = jnp.pad(W3, ((0, 0), (0, 126)))
  b3p = jnp.pad(b3, (0, 126))

  g1 = _pre_call(x, W1, dinv_col)
  p1 = _agg128(g1, row, col, ew)[:, :N]
  g2 = _mid_call(p1, g1, dinv_col, b1.reshape(1, 128), W2)
  p2 = _agg128(g2, row, col, ew)[:, :N]
  g3 = _mid_call(p2, g2, dinv_col, b2.reshape(1, 128), W3p)
  p3 = _agg128(g3, row, col, ew)[:, :N]
  out = _post_call(p3, g3, dinv_col, b3p.reshape(1, 128))
  return out[:, :2]


# speedup vs baseline: 19.8054x; 1.1436x over previous
import functools

import jax
import jax.numpy as jnp
from jax import lax
from jax.experimental import pallas as pl
from jax.experimental.pallas import tpu as pltpu
from jax.experimental.pallas import tpu_sc as plsc

N = 10000
E = 320000
NC = 2
NS = 16
NW = NC * NS
EPW = E // NW
K = 80
NCH = EPW // K
TAIL = 16
NP = 10240
RPT = NP // NS

_mesh = plsc.VectorSubcoreMesh(
    core_axis_name="c", subcore_axis_name="s", num_cores=NC, num_subcores=NS)
_sc_params = pltpu.CompilerParams(needs_layout_passes=False)


def _deg_body(col_h, ew_h, degp_h, col_v, ew_v, deg_v, sem):
  c = lax.axis_index("c")
  s = lax.axis_index("s")
  wid = s * NC + c
  base = wid * EPW
  cp = pltpu.async_copy(col_h.at[pl.ds(base, EPW)], col_v, sem)
  cp2 = pltpu.async_copy(ew_h.at[pl.ds(base, EPW)], ew_v, sem)

  def zero(i, _):
    deg_v[pl.ds(i * 16, 16)] = jnp.zeros((16,), jnp.float32)
    return 0
  lax.fori_loop(0, N // 16, zero, 0)
  cp.wait()
  cp2.wait()

  def acc(i, _):
    idx = col_v[pl.ds(i * 16, 16)]
    w = ew_v[pl.ds(i * 16, 16)]
    plsc.addupdate_scatter(deg_v, [idx], w)
    return 0
  lax.fori_loop(0, EPW // 16, acc, 0)
  pltpu.sync_copy(deg_v, degp_h.at[pl.ds(wid * N, N)])


_deg_call = pl.kernel(
    _deg_body,
    out_type=jax.ShapeDtypeStruct((NW * N,), jnp.float32),
    mesh=_mesh,
    scratch_types=[
        pltpu.VMEM((EPW,), jnp.int32),
        pltpu.VMEM((EPW,), jnp.float32),
        pltpu.VMEM((N,), jnp.float32),
        pltpu.SemaphoreType.DMA,
    ],
    compiler_params=_sc_params,
)


def _agg_body(d, g_h, row_h, col_h, ew_h, out_h,
              col_a, ew_a, rbig, c0i, c1i,
              rows0, rows1, acc, g0s, g1s, s0s, s1s):
  c = lax.axis_index("c")
  s = lax.axis_index("s")
  wid = s * NC + c
  r0 = s * RPT
  nj = d // 16
  base = wid * EPW

  cpb = pltpu.async_copy(col_h.at[pl.ds(base, EPW)], col_a, g1s)
  cpc = pltpu.async_copy(ew_h.at[pl.ds(base, EPW)], ew_a, s0s)

  def zrow(r, _):
    for j in range(nj):
      rows0[r, pl.ds(j * 16, 16)] = jnp.zeros((16,), jnp.float32)
    return 0
  lax.fori_loop(0, K, zrow, 0)

  def zcp(k, _):
    pltpu.sync_copy(rows0, acc.at[pl.ds(r0 + k * K, K)])
    return 0
  lax.fori_loop(0, RPT // K, zcp, 0)
  cpb.wait()
  cpc.wait()
  plsc.subcore_barrier()

  def scale(i, n, civ, rows_v):
    def group(gi, _):
      e0 = i * K + gi * 16
      civ[pl.ds(gi * 16, 16)] = col_a[pl.ds(e0, 16)]
      wv = ew_a[pl.ds(e0, 16)]
      for lane in range(16):
        bv = jnp.full((16,), wv[lane], jnp.float32)
        e = gi * 16 + lane
        for j in range(nj):
          sl = pl.ds(j * 16, 16)
          rows_v[e, sl] = rows_v[e, sl] * bv
      return 0
    lax.fori_loop(0, n // 16, group, 0)

  def drain(rows_v, sem):
    pltpu.make_async_copy(g_h.at[pl.ds(0, K)], rows_v, sem).wait()

  def pair(i0, roff):
    i1 = i0 + 1

    @pl.when(i0 > 0)
    def _():
      drain(rows0, s0s)
    cp0 = pltpu.async_copy(g_h.at[rbig.at[pl.ds(roff, K)]], rows0, g0s)

    @pl.when(i0 > 0)
    def _():
      drain(rows1, s1s)
    cp1 = pltpu.async_copy(g_h.at[rbig.at[pl.ds(roff + K, K)]], rows1, g1s)
    cp0.wait()
    scale(i0, K, c0i, rows0)
    pltpu.async_copy(rows0, acc.at[c0i], s0s, add=True)
    cp1.wait()
    scale(i1, K, c1i, rows1)
    pltpu.async_copy(rows1, acc.at[c1i], s1s, add=True)

  def group10(g, _):
    pltpu.sync_copy(row_h.at[pl.ds(base + g * 10 * K, 10 * K)], rbig)
    def p5(p, _):
      pair(g * 10 + 2 * p, 2 * p * K)
      return 0
    lax.fori_loop(0, 5, p5, 0)
    return 0
  lax.fori_loop(0, NCH // 10, group10, 0)

  tbase = (NCH // 10) * 10
  pltpu.sync_copy(row_h.at[pl.ds(base + tbase * K, 5 * K)],
                  rbig.at[pl.ds(0, 5 * K)])
  pair(tbase, 0)
  pair(tbase + 2, 2 * K)
  drain(rows0, s0s)
  pltpu.async_copy(g_h.at[rbig.at[pl.ds(4 * K, K)]], rows0, g0s).wait()
  scale(tbase + 4, K, c0i, rows0)
  pltpu.sync_copy(rows0, acc.at[c0i], add=True)
  drain(rows1, s1s)

  plsc.subcore_barrier()
  pltpu.sync_copy(acc.at[pl.ds(r0, RPT)], out_h.at[c, pl.ds(r0, RPT)])


def _make_agg(d):
  return pl.kernel(
      functools.partial(_agg_body, d),
      out_type=jax.ShapeDtypeStruct((NC, NP, d), jnp.float32),
      mesh=_mesh,
      scratch_types=[
          pltpu.VMEM((EPW,), jnp.int32),
          pltpu.VMEM((EPW,), jnp.float32),
          pltpu.VMEM((10 * K,), jnp.int32),
          pltpu.VMEM((K,), jnp.int32),
          pltpu.VMEM((K,), jnp.int32),
          pltpu.VMEM((K, d), jnp.float32),
          pltpu.VMEM((K, d), jnp.float32),
          pltpu.VMEM_SHARED((NP, d), jnp.float32),
          pltpu.SemaphoreType.DMA,
          pltpu.SemaphoreType.DMA,
          pltpu.SemaphoreType.DMA,
          pltpu.SemaphoreType.DMA,
      ],
      compiler_params=_sc_params,
  )


_agg128 = _make_agg(128)


_BM = 1000


def _dinv_body(degp_ref, out_ref):
  s = jnp.sum(degp_ref[...], axis=0, keepdims=True)
  out_ref[...] = lax.rsqrt(1.0 + s)


def _dinv_call(degp):
  return pl.pallas_call(
      _dinv_body,
      out_shape=jax.ShapeDtypeStruct((1, N), jnp.float32),
  )(degp)


def _pre_body(x_ref, w_ref, dinv_ref, o_ref):
  h = jnp.dot(x_ref[...], w_ref[...], preferred_element_type=jnp.float32)
  o_ref[...] = dinv_ref[...] * h


def _pre_call(x, w, dinv_col):
  dout = w.shape[1]
  return pl.pallas_call(
      _pre_body,
      grid=(N // _BM,),
      in_specs=[
          pl.BlockSpec((_BM, 128), lambda i: (i, 0)),
          pl.BlockSpec((128, dout), lambda i: (0, 0)),
          pl.BlockSpec((_BM, 1), lambda i: (i, 0)),
      ],
      out_specs=pl.BlockSpec((_BM, dout), lambda i: (i, 0)),
      out_shape=jax.ShapeDtypeStruct((N, dout), jnp.float32),
  )(x, w, dinv_col)


def _mid_body(p_ref, g_ref, dinv_ref, b_ref, w_ref, o_ref):
  t = p_ref[0] + p_ref[1] + g_ref[...]
  xn = jnp.maximum(dinv_ref[...] * t + b_ref[...], 0.0)
  h = jnp.dot(xn, w_ref[...], preferred_element_type=jnp.float32)
  o_ref[...] = dinv_ref[...] * h


def _mid_call(p, g, dinv_col, b_row, w):
  din = g.shape[1]
  dout = w.shape[1]
  return pl.pallas_call(
      _mid_body,
      grid=(N // _BM,),
      in_specs=[
          pl.BlockSpec((NC, _BM, din), lambda i: (0, i, 0)),
          pl.BlockSpec((_BM, din), lambda i: (i, 0)),
          pl.BlockSpec((_BM, 1), lambda i: (i, 0)),
          pl.BlockSpec((1, din), lambda i: (0, 0)),
          pl.BlockSpec((din, dout), lambda i: (0, 0)),
      ],
      out_specs=pl.BlockSpec((_BM, dout), lambda i: (i, 0)),
      out_shape=jax.ShapeDtypeStruct((N, dout), jnp.float32),
  )(p, g, dinv_col, b_row, w)


def _post_body(p_ref, g_ref, dinv_ref, b_ref, o_ref):
  t = p_ref[0] + p_ref[1] + g_ref[...]
  o_ref[...] = dinv_ref[...] * t + b_ref[...]


def _post_call(p, g, dinv_col, b_row):
  d = g.shape[1]
  return pl.pallas_call(
      _post_body,
      grid=(N // _BM,),
      in_specs=[
          pl.BlockSpec((NC, _BM, d), lambda i: (0, i, 0)),
          pl.BlockSpec((_BM, d), lambda i: (i, 0)),
          pl.BlockSpec((_BM, 1), lambda i: (i, 0)),
          pl.BlockSpec((1, d), lambda i: (0, 0)),
      ],
      out_specs=pl.BlockSpec((_BM, d), lambda i: (i, 0)),
      out_shape=jax.ShapeDtypeStruct((N, d), jnp.float32),
  )(p, g, dinv_col, b_row)


def kernel(x, edge_index, edge_attr, W1, b1, W2, b2, W3, b3):
  row = edge_index[0].astype(jnp.int32)
  col = edge_index[1].astype(jnp.int32)
  ew = edge_attr.astype(jnp.float32)

  degp = _deg_call(col, ew).reshape(NW, N)
  dinv_col = _dinv_call(degp).reshape(N, 1)

  W3p = jnp.pad(W3, ((0, 0), (0, 126)))
  b3p = jnp.pad(b3, (0, 126))

  g1 = _pre_call(x, W1, dinv_col)
  p1 = _agg128(g1, row, col, ew)[:, :N]
  g2 = _mid_call(p1, g1, dinv_col, b1.reshape(1, 128), W2)
  p2 = _agg128(g2, row, col, ew)[:, :N]
  g3 = _mid_call(p2, g2, dinv_col, b2.reshape(1, 128), W3p)
  p3 = _agg128(g3, row, col, ew)[:, :N]
  out = _post_call(p3, g3, dinv_col, b3p.reshape(1, 128))
  return out[:, :2]


# speedup vs baseline: 19.8449x; 1.0020x over previous
import functools

import jax
import jax.numpy as jnp
from jax import lax
from jax.experimental import pallas as pl
from jax.experimental.pallas import tpu as pltpu
from jax.experimental.pallas import tpu_sc as plsc

N = 10000
E = 320000
NC = 2
NS = 16
NW = NC * NS
EPW = E // NW
K = 128
NCH = EPW // K
TAIL = EPW - NCH * K
G6 = 6
NP = 10240
RPT = NP // NS

_mesh = plsc.VectorSubcoreMesh(
    core_axis_name="c", subcore_axis_name="s", num_cores=NC, num_subcores=NS)
_sc_params = pltpu.CompilerParams(needs_layout_passes=False)


def _deg_body(col_h, ew_h, degp_h, col_v, ew_v, deg_v, sem):
  c = lax.axis_index("c")
  s = lax.axis_index("s")
  wid = s * NC + c
  base = wid * EPW
  cp = pltpu.async_copy(col_h.at[pl.ds(base, EPW)], col_v, sem)
  cp2 = pltpu.async_copy(ew_h.at[pl.ds(base, EPW)], ew_v, sem)

  def zero(i, _):
    deg_v[pl.ds(i * 16, 16)] = jnp.zeros((16,), jnp.float32)
    return 0
  lax.fori_loop(0, N // 16, zero, 0)
  cp.wait()
  cp2.wait()

  def acc(i, _):
    idx = col_v[pl.ds(i * 16, 16)]
    w = ew_v[pl.ds(i * 16, 16)]
    plsc.addupdate_scatter(deg_v, [idx], w)
    return 0
  lax.fori_loop(0, EPW // 16, acc, 0)
  pltpu.sync_copy(deg_v, degp_h.at[pl.ds(wid * N, N)])


_deg_call = pl.kernel(
    _deg_body,
    out_type=jax.ShapeDtypeStruct((NW * N,), jnp.float32),
    mesh=_mesh,
    scratch_types=[
        pltpu.VMEM((EPW,), jnp.int32),
        pltpu.VMEM((EPW,), jnp.float32),
        pltpu.VMEM((N,), jnp.float32),
        pltpu.SemaphoreType.DMA,
    ],
    compiler_params=_sc_params,
)


def _agg_body(d, g_h, row_h, col_h, ew_h, out_h,
              col_a, ewbig, rbig, c0i, c1i, rti, cti, ewt, rowst,
              rows0, rows1, acc, g0s, g1s, s0s, s1s):
  c = lax.axis_index("c")
  s = lax.axis_index("s")
  wid = s * NC + c
  r0 = s * RPT
  nj = d // 16
  base = wid * EPW

  cpb = pltpu.async_copy(col_h.at[pl.ds(base, EPW)], col_a, g1s)

  def zrow(r, _):
    for j in range(nj):
      rows0[r, pl.ds(j * 16, 16)] = jnp.zeros((16,), jnp.float32)
    return 0
  lax.fori_loop(0, K, zrow, 0)

  def zcp(k, _):
    pltpu.sync_copy(rows0, acc.at[pl.ds(r0 + k * K, K)])
    return 0
  lax.fori_loop(0, RPT // K, zcp, 0)
  cpb.wait()
  plsc.subcore_barrier()

  def scale(i, woff, n, civ, rows_v):
    def group(gi, _):
      e0 = i * K + gi * 16
      civ[pl.ds(gi * 16, 16)] = col_a[pl.ds(e0, 16)]
      wv = ewbig[pl.ds(woff + gi * 16, 16)]
      for lane in range(16):
        bv = jnp.full((16,), wv[lane], jnp.float32)
        e = gi * 16 + lane
        for j in range(nj):
          sl = pl.ds(j * 16, 16)
          rows_v[e, sl] = rows_v[e, sl] * bv
      return 0
    lax.fori_loop(0, n // 16, group, 0)

  def drain(rows_v, sem):
    pltpu.make_async_copy(g_h.at[pl.ds(0, K)], rows_v, sem).wait()

  def pair(i0, roff):
    i1 = i0 + 1

    @pl.when(i0 > 0)
    def _():
      drain(rows0, s0s)
    cp0 = pltpu.async_copy(g_h.at[rbig.at[pl.ds(roff, K)]], rows0, g0s)

    @pl.when(i0 > 0)
    def _():
      drain(rows1, s1s)
    cp1 = pltpu.async_copy(g_h.at[rbig.at[pl.ds(roff + K, K)]], rows1, g1s)
    cp0.wait()
    scale(i0, roff, K, c0i, rows0)
    pltpu.async_copy(rows0, acc.at[c0i], s0s, add=True)
    cp1.wait()
    scale(i1, roff + K, K, c1i, rows1)
    pltpu.async_copy(rows1, acc.at[c1i], s1s, add=True)

  def group6(g, _):
    pltpu.sync_copy(row_h.at[pl.ds(base + g * G6 * K, G6 * K)], rbig)
    pltpu.sync_copy(ew_h.at[pl.ds(base + g * G6 * K, G6 * K)], ewbig)
    def p3(p, _):
      pair(g * G6 + 2 * p, 2 * p * K)
      return 0
    lax.fori_loop(0, G6 // 2, p3, 0)
    return 0
  lax.fori_loop(0, NCH // G6, group6, 0)

  tb = base + NCH * K
  pltpu.sync_copy(row_h.at[pl.ds(tb, TAIL)], rti)
  pltpu.sync_copy(ew_h.at[pl.ds(tb, TAIL)], ewt)
  cpt = pltpu.async_copy(g_h.at[rti], rowst, g0s)
  cti[pl.ds(0, 16)] = col_a[pl.ds(NCH * K, 16)]
  wv = ewt[pl.ds(0, 16)]
  cpt.wait()
  for lane in range(16):
    bv = jnp.full((16,), wv[lane], jnp.float32)
    for j in range(nj):
      sl = pl.ds(j * 16, 16)
      rowst[lane, sl] = rowst[lane, sl] * bv
  pltpu.sync_copy(rowst, acc.at[cti], add=True)
  drain(rows0, s0s)
  drain(rows1, s1s)

  plsc.subcore_barrier()
  pltpu.sync_copy(acc.at[pl.ds(r0, RPT)], out_h.at[c, pl.ds(r0, RPT)])


def _make_agg(d):
  return pl.kernel(
      functools.partial(_agg_body, d),
      out_type=jax.ShapeDtypeStruct((NC, NP, d), jnp.float32),
      mesh=_mesh,
      scratch_types=[
          pltpu.VMEM((EPW,), jnp.int32),
          pltpu.VMEM((G6 * K,), jnp.float32),
          pltpu.VMEM((G6 * K,), jnp.int32),
          pltpu.VMEM((K,), jnp.int32),
          pltpu.VMEM((K,), jnp.int32),
          pltpu.VMEM((TAIL,), jnp.int32),
          pltpu.VMEM((TAIL,), jnp.int32),
          pltpu.VMEM((TAIL,), jnp.float32),
          pltpu.VMEM((TAIL, d), jnp.float32),
          pltpu.VMEM((K, d), jnp.float32),
          pltpu.VMEM((K, d), jnp.float32),
          pltpu.VMEM_SHARED((NP, d), jnp.float32),
          pltpu.SemaphoreType.DMA,
          pltpu.SemaphoreType.DMA,
          pltpu.SemaphoreType.DMA,
          pltpu.SemaphoreType.DMA,
      ],
      compiler_params=_sc_params,
  )


_agg128 = _make_agg(128)


_BM = 1000


def _dinv_body(degp_ref, out_ref):
  s = jnp.sum(degp_ref[...], axis=0, keepdims=True)
  out_ref[...] = lax.rsqrt(1.0 + s)


def _dinv_call(degp):
  return pl.pallas_call(
      _dinv_body,
      out_shape=jax.ShapeDtypeStruct((1, N), jnp.float32),
  )(degp)


def _pre_body(x_ref, w_ref, dinv_ref, o_ref):
  h = jnp.dot(x_ref[...], w_ref[...], preferred_element_type=jnp.float32)
  o_ref[...] = dinv_ref[...] * h


def _pre_call(x, w, dinv_col):
  dout = w.shape[1]
  return pl.pallas_call(
      _pre_body,
      grid=(N // _BM,),
      in_specs=[
          pl.BlockSpec((_BM, 128), lambda i: (i, 0)),
          pl.BlockSpec((128, dout), lambda i: (0, 0)),
          pl.BlockSpec((_BM, 1), lambda i: (i, 0)),
      ],
      out_specs=pl.BlockSpec((_BM, dout), lambda i: (i, 0)),
      out_shape=jax.ShapeDtypeStruct((N, dout), jnp.float32),
  )(x, w, dinv_col)


def _mid_body(p_ref, g_ref, dinv_ref, b_ref, w_ref, o_ref):
  t = p_ref[0] + p_ref[1] + g_ref[...]
  xn = jnp.maximum(dinv_ref[...] * t + b_ref[...], 0.0)
  h = jnp.dot(xn, w_ref[...], preferred_element_type=jnp.float32)
  o_ref[...] = dinv_ref[...] * h


def _mid_call(p, g, dinv_col, b_row, w):
  din = g.shape[1]
  dout = w.shape[1]
  return pl.pallas_call(
      _mid_body,
      grid=(N // _BM,),
      in_specs=[
          pl.BlockSpec((NC, _BM, din), lambda i: (0, i, 0)),
          pl.BlockSpec((_BM, din), lambda i: (i, 0)),
          pl.BlockSpec((_BM, 1), lambda i: (i, 0)),
          pl.BlockSpec((1, din), lambda i: (0, 0)),
          pl.BlockSpec((din, dout), lambda i: (0, 0)),
      ],
      out_specs=pl.BlockSpec((_BM, dout), lambda i: (i, 0)),
      out_shape=jax.ShapeDtypeStruct((N, dout), jnp.float32),
  )(p, g, dinv_col, b_row, w)


def _post_body(p_ref, g_ref, dinv_ref, b_ref, o_ref):
  t = p_ref[0] + p_ref[1] + g_ref[...]
  o_ref[...] = dinv_ref[...] * t + b_ref[...]


def _post_call(p, g, dinv_col, b_row):
  d = g.shape[1]
  return pl.pallas_call(
      _post_body,
      grid=(N // _BM,),
      in_specs=[
          pl.BlockSpec((NC, _BM, d), lambda i: (0, i, 0)),
          pl.BlockSpec((_BM, d), lambda i: (i, 0)),
          pl.BlockSpec((_BM, 1), lambda i: (i, 0)),
          pl.BlockSpec((1, d), lambda i: (0, 0)),
      ],
      out_specs=pl.BlockSpec((_BM, d), lambda i: (i, 0)),
      out_shape=jax.ShapeDtypeStruct((N, d), jnp.float32),
  )(p, g, dinv_col, b_row)


def kernel(x, edge_index, edge_attr, W1, b1, W2, b2, W3, b3):
  row = edge_index[0].astype(jnp.int32)
  col = edge_index[1].astype(jnp.int32)
  ew = edge_attr.astype(jnp.float32)

  degp = _deg_call(col, ew).reshape(NW, N)
  dinv_col = _dinv_call(degp).reshape(N, 1)

  W3p = jnp.pad(W3, ((0, 0), (0, 126)))
  b3p = jnp.pad(b3, (0, 126))

  g1 = _pre_call(x, W1, dinv_col)
  p1 = _agg128(g1, row, col, ew)[:, :N]
  g2 = _mid_call(p1, g1, dinv_col, b1.reshape(1, 128), W2)
  p2 = _agg128(g2, row, col, ew)[:, :N]
  g3 = _mid_call(p2, g2, dinv_col, b2.reshape(1, 128), W3p)
  p3 = _agg128(g3, row, col, ew)[:, :N]
  out = _post_call(p3, g3, dinv_col, b3p.reshape(1, 128))
  return out[:, :2]
